# baseline (speedup 1.0000x reference)
import functools

import jax
import jax.numpy as jnp
from jax import lax
from jax.experimental import pallas as pl
from jax.experimental.pallas import tpu as pltpu
from jax.experimental.pallas import tpu_sc as plsc

_BATCH = 16384
_HIST = 20
_D = 64
_NC = 2
_NS = 16
_NW = _NC * _NS
_ROWS_PAD = 1000064
_NT = _ROWS_PAD // 128
_PR = _ROWS_PAD // 2
_BPW = _BATCH // _NW
_GRP = 32
_NG = _BPW // _GRP


def _make_transpose():
    mesh = plsc.VectorSubcoreMesh(core_axis_name="c", subcore_axis_name="s")

    @functools.partial(
        pl.kernel,
        mesh=mesh,
        compiler_params=pltpu.CompilerParams(needs_layout_passes=False),
        out_type=jax.ShapeDtypeStruct((_PR, 128), jnp.float32),
        scratch_types=[
            pltpu.VMEM((2, 64, 129), jnp.float32),
            pltpu.VMEM((2, 64, 128), jnp.float32),
            pltpu.SemaphoreType.DMA,
            pltpu.SemaphoreType.DMA,
            pltpu.SemaphoreType.DMA,
            pltpu.SemaphoreType.DMA,
        ],
    )
    def transpose(tabT_hbm, tl_hbm, vin, vout, gs0, gs1, ss0, ss1):
        gsem = (gs0, gs1)
        ssem = (ss0, ss1)
        wid = lax.axis_index("s") * _NC + lax.axis_index("c")
        lanes = lax.iota(jnp.int32, 16)
        rowsel = [j0 + lanes for j0 in range(0, 64, 16)]
        nitems = 246

        def tc_of(it):
            return jnp.minimum(it * _NW + wid, _NT - 1)

        def fire_in(it, b):
            pltpu.async_copy(
                tabT_hbm.at[:, pl.ds(tc_of(it) * 128, 128)],
                vin.at[b, :, pl.ds(0, 128)],
                gsem[b],
            )

        def wait_in(b):
            pltpu.make_async_copy(
                tabT_hbm.at[:, pl.ds(0, 128)],
                vin.at[b, :, pl.ds(0, 128)],
                gsem[b],
            ).wait()

        def fire_store(it, b):
            pltpu.async_copy(
                vout.at[b], tl_hbm.at[pl.ds(tc_of(it) * 64, 64)], ssem[b]
            )

        def wait_store(b):
            pltpu.make_async_copy(
                tabT_hbm.at[:, pl.ds(0, 128)], vout.at[b], ssem[b]
            ).wait()

        def compute(b):
            vin_b = vin.at[b]
            vout_b = vout.at[b]

            @plsc.parallel_loop(0, 64, unroll=4)
            def rowq(q):
                c0 = jnp.full((16,), 2 * q, jnp.int32)
                c1 = c0 + 1
                vs = []
                for h, cv in ((0, c0), (1, c1)):
                    for k in range(4):
                        vs.append((h, k, plsc.load_gather(vin_b, [rowsel[k], cv])))
                for h, k, v in vs:
                    vout_b[q, pl.ds(h * 64 + k * 16, 16)] = v

        for b in range(2):
            fire_in(b, b)

        def body(i, carry):
            for b in range(2):
                it = i * 2 + b
                wait_in(b)
                pl.when(it >= 2)(lambda: wait_store(b))
                compute(b)
                fire_in(it + 2, b)
                fire_store(it, b)
            return carry

        lax.fori_loop(0, nitems // 2, body, 0)

        for b in range(2):
            wait_in(b)
            wait_store(b)

    return transpose


def _make_gather():
    mesh = plsc.VectorSubcoreMesh(core_axis_name="c", subcore_axis_name="s")

    @functools.partial(
        pl.kernel,
        mesh=mesh,
        compiler_params=pltpu.CompilerParams(use_tc_tiling_on_sc=False),
        out_type=jax.ShapeDtypeStruct((_BATCH, _HIST, _D), jnp.float32),
        scratch_types=[
            pltpu.VMEM((_BPW, _HIST), jnp.int32),
            pltpu.VMEM((2, _GRP, _HIST, _D), jnp.float32),
            pltpu.SemaphoreType.DMA,
            pltpu.SemaphoreType.DMA,
            pltpu.SemaphoreType.DMA,
            pltpu.SemaphoreType.DMA,
        ],
    )
    def gather(idx_hbm, tl_hbm, out_hbm, idxv, rows, gs0, gs1, ss0, ss1):
        gsem = (gs0, gs1)
        ssem = (ss0, ss1)
        wid = lax.axis_index("s") * _NC + lax.axis_index("c")
        base = wid * _BPW
        pltpu.sync_copy(idx_hbm.at[wid], idxv)

        def fire(g, b):
            for k in range(_GRP):
                pltpu.async_copy(
                    tl_hbm.at[idxv.at[g * _GRP + k]], rows.at[b, k], gsem[b]
                )

        def drain(g, b):
            for k in range(_GRP):
                pltpu.make_async_copy(
                    tl_hbm.at[idxv.at[g * _GRP + k]], rows.at[b, k], gsem[b]
                ).wait()

        def store(g, b):
            pltpu.async_copy(
                rows.at[b], out_hbm.at[pl.ds(base + g * _GRP, _GRP)], ssem[b]
            )

        def wait_store(g, b):
            pltpu.make_async_copy(
                rows.at[b], out_hbm.at[pl.ds(base + g * _GRP, _GRP)], ssem[b]
            ).wait()

        for b in range(2):
            fire(b, b)

        def body(i, carry):
            g0 = i * 2
            for b in range(2):
                g = g0 + b
                drain(g, b)
                store(g, b)
                wait_store(g, b)
                fire(g + 2, b)
            return carry

        lax.fori_loop(0, (_NG - 2) // 2, body, 0)

        for b in range(2):
            g = _NG - 2 + b
            drain(g, b)
            store(g, b)
            wait_store(g, b)

    return gather


_transpose = _make_transpose()
_gather = _make_gather()


def kernel(x, table):
    tl = _transpose(table.T)
    tl2 = tl.reshape(_ROWS_PAD, _D)
    idx = x.reshape(_NW, _BPW, _HIST).astype(jnp.int32)
    return _gather(idx, tl2)


# speedup vs baseline: 1.4962x; 1.4962x over previous
import functools

import jax
import jax.numpy as jnp
from jax import lax
from jax.experimental import pallas as pl
from jax.experimental.pallas import tpu as pltpu
from jax.experimental.pallas import tpu_sc as plsc

_BATCH = 16384
_HIST = 20
_D = 64
_B = _BATCH * _HIST
_NC = 2
_NS = 16
_NW = _NC * _NS
_CHUNK = 128
_PERW = _B // _NW
_CH = _PERW // _CHUNK
_K = 4
_SUPER = _K * _CHUNK
_S = _CH // _K
_NBUF = 2


def _make_gather():
    mesh = plsc.VectorSubcoreMesh(core_axis_name="c", subcore_axis_name="s")

    @functools.partial(
        pl.kernel,
        mesh=mesh,
        compiler_params=pltpu.CompilerParams(use_tc_tiling_on_sc=False),
        out_type=jax.ShapeDtypeStruct((_B, _D), jnp.float32),
        scratch_types=[
            pltpu.VMEM((_CH, _CHUNK), jnp.int32),
            pltpu.VMEM((_NBUF, _SUPER, _D), jnp.float32),
            pltpu.SemaphoreType.DMA,
            pltpu.SemaphoreType.DMA,
            pltpu.SemaphoreType.DMA,
            pltpu.SemaphoreType.DMA,
        ],
    )
    def gather(idx_hbm, table_hbm, out_hbm, idx_v, rows, gs0, gs1, ss0, ss1):
        gsem = (gs0, gs1)
        ssem = (ss0, ss1)
        wid = lax.axis_index("s") * _NC + lax.axis_index("c")
        base = wid * _PERW
        pltpu.sync_copy(idx_hbm.at[wid], idx_v)

        def fire(s, b):
            for j in range(_K):
                pltpu.async_copy(
                    table_hbm.at[idx_v.at[s * _K + j]],
                    rows.at[b, pl.ds(j * _CHUNK, _CHUNK)],
                    gsem[b],
                )

        def wait_gather(b):
            pltpu.make_async_copy(
                table_hbm.at[pl.ds(0, _SUPER)], rows.at[b], gsem[b]
            ).wait()

        def store(s, b):
            pltpu.async_copy(
                rows.at[b], out_hbm.at[pl.ds(base + s * _SUPER, _SUPER)], ssem[b]
            )

        def wait_store(b):
            pltpu.make_async_copy(
                table_hbm.at[pl.ds(0, _SUPER)], rows.at[b], ssem[b]
            ).wait()

        for b in range(_NBUF):
            fire(b, b)

        def body(i, carry):
            s0 = i * _NBUF
            for b in range(_NBUF):
                s = s0 + b
                wait_gather(b)
                store(s, b)
                wait_store(b)
                fire(s + _NBUF, b)
            return carry

        lax.fori_loop(0, (_S - _NBUF) // _NBUF, body, 0)

        for b in range(_NBUF):
            s = _S - _NBUF + b
            wait_gather(b)
            store(s, b)
            wait_store(b)

    return gather


_gather = _make_gather()


def kernel(x, table):
    idx = x.reshape(_NW, _CH, _CHUNK).astype(jnp.int32)
    out = _gather(idx, table)
    return out.reshape(_BATCH, _HIST, _D)
